# Initial kernel scaffold; baseline (speedup 1.0000x reference)
import jax
import jax.numpy as jnp
from jax.experimental import pallas as pl


def kernel(x, pt_indices, pt_values, pd_indices, pd_values, W, b):
    raise NotImplementedError("write your pallas kernel here")


# speedup vs baseline: 3.4948x; 3.4948x over previous
import functools

import jax
import jax.numpy as jnp
from jax import lax
from jax.experimental import pallas as pl
from jax.experimental.pallas import tpu as pltpu
from jax.experimental.pallas import tpu_sc as plsc

NC = 2
NS = 16
NW = NC * NS
K = 128
L = 16


def _matmul(x, W):
    n, d_in = x.shape
    d_out = W.shape[1]
    blk = 1000

    def mm(x_ref, w_ref, o_ref):
        o_ref[...] = jnp.dot(x_ref[...], w_ref[...],
                             preferred_element_type=jnp.float32)

    return pl.pallas_call(
        mm,
        grid=(n // blk,),
        in_specs=[pl.BlockSpec((blk, d_in), lambda i: (i, 0)),
                  pl.BlockSpec((d_in, d_out), lambda i: (0, 0))],
        out_specs=pl.BlockSpec((blk, d_out), lambda i: (i, 0)),
        out_shape=jax.ShapeDtypeStruct((n, d_out), jnp.float32),
    )(x, W)


def _spmm_partials(rows, cols, vals, dense, n_pad, d):
    e_pad = rows.shape[0]
    epw = e_pad // NW
    n_chunks = epw // K
    npt = n_pad // NS
    dv = d // L

    mesh = plsc.VectorSubcoreMesh(core_axis_name="c", subcore_axis_name="s")

    @functools.partial(
        pl.kernel,
        mesh=mesh,
        out_type=jax.ShapeDtypeStruct((NC, n_pad, d), jnp.float32),
        scratch_types=[
            pltpu.VMEM((K,), jnp.int32),
            pltpu.VMEM((K,), jnp.int32),
            pltpu.VMEM((K,), jnp.float32),
            pltpu.VMEM((K, d), jnp.float32),
            pltpu.VMEM_SHARED((n_pad, d), jnp.float32),
            pltpu.SemaphoreType.DMA,
        ],
    )
    def spmm(rows_hbm, cols_hbm, vals_hbm, dense_hbm, out_hbm,
             col_v, row_v, val_v, rows_buf, acc, sem):
        cid = lax.axis_index("c")
        sid = lax.axis_index("s")
        wid = cid * NS + sid

        def zero_row(r, carry):
            for j in range(dv):
                rows_buf[r, pl.ds(j * L, L)] = jnp.zeros((L,), jnp.float32)
            return carry
        lax.fori_loop(0, K, zero_row, 0)
        for j in range(npt // K):
            pltpu.sync_copy(rows_buf, acc.at[pl.ds(sid * npt + j * K, K)])
        plsc.subcore_barrier()

        base = wid * epw

        def chunk_body(c, carry):
            off = base + c * K
            pltpu.sync_copy(cols_hbm.at[pl.ds(off, K)], col_v)
            pltpu.sync_copy(rows_hbm.at[pl.ds(off, K)], row_v)
            pltpu.sync_copy(vals_hbm.at[pl.ds(off, K)], val_v)
            pltpu.async_copy(dense_hbm.at[col_v], rows_buf, sem).wait()

            def scale_grp(g, carry2):
                vgrp = val_v[pl.ds(g * L, L)]
                for i in range(L):
                    vv = vgrp[i]
                    r = g * L + i
                    for j in range(dv):
                        rows_buf[r, pl.ds(j * L, L)] = (
                            rows_buf[r, pl.ds(j * L, L)] * vv)
                return carry2
            lax.fori_loop(0, K // L, scale_grp, 0)

            pltpu.sync_copy(rows_buf, acc.at[row_v], add=True)
            return carry
        lax.fori_loop(0, n_chunks, chunk_body, 0)

        plsc.subcore_barrier()
        pltpu.sync_copy(acc.at[pl.ds(sid * npt, npt)],
                        out_hbm.at[cid, pl.ds(sid * npt, npt)])

    return spmm(rows, cols, vals, dense)


def _combine(partials, bias, n_rows):
    d = partials.shape[-1]
    blk = 1000
    assert n_rows % blk == 0

    if bias is None:
        def body(p_ref, o_ref):
            o_ref[...] = p_ref[0] + p_ref[1]
        in_specs = [pl.BlockSpec((NC, blk, d), lambda i: (0, i, 0))]
        operands = (partials,)
    else:
        def body(p_ref, b_ref, o_ref):
            o_ref[...] = p_ref[0] + p_ref[1] + b_ref[...]
        in_specs = [pl.BlockSpec((NC, blk, d), lambda i: (0, i, 0)),
                    pl.BlockSpec((1, d), lambda i: (0, 0))]
        operands = (partials, bias.reshape(1, d))

    return pl.pallas_call(
        body,
        grid=(n_rows // blk,),
        in_specs=in_specs,
        out_specs=pl.BlockSpec((blk, d), lambda i: (i, 0)),
        out_shape=jax.ShapeDtypeStruct((n_rows, d), jnp.float32),
    )(*operands)


def _pad_edges(indices, values, e_pad):
    e = values.shape[0]
    if e == e_pad:
        return indices[0], indices[1], values
    pad = e_pad - e
    rows = jnp.concatenate([indices[0], jnp.zeros((pad,), jnp.int32)])
    cols = jnp.concatenate([indices[1], jnp.zeros((pad,), jnp.int32)])
    vals = jnp.concatenate([values, jnp.zeros((pad,), jnp.float32)])
    return rows, cols, vals


def kernel(x, pt_indices, pt_values, pd_indices, pd_values, W, b):
    n, _ = x.shape
    d = W.shape[1]
    e = pt_values.shape[0]

    grain = NW * K
    e_pad = -(-e // grain) * grain
    n_pad = -(-n // (NS * K)) * (NS * K)

    pt_rows, pt_cols, pt_vals = _pad_edges(pt_indices, pt_values, e_pad)
    pd_rows, pd_cols, pd_vals = _pad_edges(pd_indices, pd_values, e_pad)

    support = _matmul(x, W)
    p1 = _spmm_partials(pt_rows, pt_cols, pt_vals, support, n_pad, d)
    midpu = _combine(p1, None, n)
    p2 = _spmm_partials(pd_rows, pd_cols, pd_vals, midpu, n_pad, d)
    return _combine(p2, b, n)


# speedup vs baseline: 5.6828x; 1.6261x over previous
import functools

import jax
import jax.numpy as jnp
from jax import lax
from jax.experimental import pallas as pl
from jax.experimental.pallas import tpu as pltpu
from jax.experimental.pallas import tpu_sc as plsc

NC = 2
NS = 16
NW = NC * NS
K = 64
L = 16


def _matmul(x, W):
    n, d_in = x.shape
    d_out = W.shape[1]
    blk = 1000

    def mm(x_ref, w_ref, o_ref):
        o_ref[...] = jnp.dot(x_ref[...], w_ref[...],
                             preferred_element_type=jnp.float32)

    return pl.pallas_call(
        mm,
        grid=(n // blk,),
        in_specs=[pl.BlockSpec((blk, d_in), lambda i: (i, 0)),
                  pl.BlockSpec((d_in, d_out), lambda i: (0, 0))],
        out_specs=pl.BlockSpec((blk, d_out), lambda i: (i, 0)),
        out_shape=jax.ShapeDtypeStruct((n, d_out), jnp.float32),
    )(x, W)


def _spmm_partials(rows, cols, vals, dense, n_pad, d):
    e_pad = rows.shape[0]
    epw = e_pad // NW
    n_chunks = epw // K
    npt = n_pad // NS
    dv = d // L

    mesh = plsc.VectorSubcoreMesh(core_axis_name="c", subcore_axis_name="s")

    @functools.partial(
        pl.kernel,
        mesh=mesh,
        out_type=jax.ShapeDtypeStruct((NC, n_pad, d), jnp.float32),
        scratch_types=[
            pltpu.VMEM((epw,), jnp.int32),
            pltpu.VMEM((epw,), jnp.int32),
            pltpu.VMEM((epw,), jnp.float32),
            pltpu.VMEM((K,), jnp.int32),
            pltpu.VMEM((K,), jnp.int32),
            pltpu.VMEM((K, d), jnp.float32),
            pltpu.VMEM((K, d), jnp.float32),
            pltpu.VMEM_SHARED((n_pad, d), jnp.float32),
            pltpu.SemaphoreType.DMA,
            pltpu.SemaphoreType.DMA,
            pltpu.SemaphoreType.DMA,
            pltpu.SemaphoreType.DMA,
        ],
    )
    def spmm(rows_hbm, cols_hbm, vals_hbm, dense_hbm, out_hbm,
             col_v, row_v, val_v, ridx0, ridx1, buf0, buf1, acc,
             gsem0, gsem1, ssem0, ssem1):
        cid = lax.axis_index("c")
        sid = lax.axis_index("s")
        wid = cid * NS + sid
        bufs = (buf0, buf1)
        ridxs = (ridx0, ridx1)
        gsems = (gsem0, gsem1)
        ssems = (ssem0, ssem1)

        def zero_row(r, carry):
            for j in range(dv):
                buf0[r, pl.ds(j * L, L)] = jnp.zeros((L,), jnp.float32)
            return carry
        lax.fori_loop(0, K, zero_row, 0)
        for j in range(npt // K):
            pltpu.sync_copy(buf0, acc.at[pl.ds(sid * npt + j * K, K)])
        plsc.subcore_barrier()

        base = wid * epw
        pltpu.sync_copy(cols_hbm.at[pl.ds(base, epw)], col_v)
        pltpu.sync_copy(rows_hbm.at[pl.ds(base, epw)], row_v)
        pltpu.sync_copy(vals_hbm.at[pl.ds(base, epw)], val_v)

        def gather(c, b):
            pltpu.async_copy(dense_hbm.at[col_v.at[pl.ds(c * K, K)]],
                             bufs[b], gsems[b])

        def scale(c, b):
            buf = bufs[b]
            ridx = ridxs[b]
            for g in range(K // L):
                ridx[pl.ds(g * L, L)] = row_v[pl.ds(c * K + g * L, L)]

            def scale_grp(g, carry):
                vgrp = val_v[pl.ds(c * K + g * L, L)]
                for i in range(L):
                    vv = vgrp[i]
                    r = g * L + i
                    for j in range(dv):
                        buf[r, pl.ds(j * L, L)] = buf[r, pl.ds(j * L, L)] * vv
                return carry
            lax.fori_loop(0, K // L, scale_grp, 0)

        def scatter(c, b):
            pltpu.async_copy(bufs[b], acc.at[ridxs[b]], ssems[b],
                             add=True)

        def wait(sem, buf):
            pltpu.make_async_copy(dense_hbm.at[pl.ds(0, K)], buf, sem).wait()

        gather(0, 0)
        gather(1, 1)
        wait(gsems[0], buf0)
        scale(0, 0)
        scatter(0, 0)

        def pair_body(i, carry):
            for b in range(2):
                c = 1 + 2 * i + b
                cb = (1 + b) % 2
                wait(ssems[1 - cb], bufs[1 - cb])
                gather(c + 1, 1 - cb)
                wait(gsems[cb], bufs[cb])
                scale(c, cb)
                scatter(c, cb)
            return carry
        lax.fori_loop(0, (n_chunks - 2) // 2, pair_body, 0)

        c_last = n_chunks - 1
        cb = c_last % 2
        wait(ssems[1 - cb], bufs[1 - cb])
        wait(gsems[cb], bufs[cb])
        scale(c_last, cb)
        scatter(c_last, cb)
        wait(ssems[cb], bufs[cb])

        plsc.subcore_barrier()
        pltpu.sync_copy(acc.at[pl.ds(sid * npt, npt)],
                        out_hbm.at[cid, pl.ds(sid * npt, npt)])

    return spmm(rows, cols, vals, dense)


def _combine(partials, bias, n_rows):
    d = partials.shape[-1]
    blk = 1000
    assert n_rows % blk == 0

    if bias is None:
        def body(p_ref, o_ref):
            o_ref[...] = p_ref[0] + p_ref[1]
        in_specs = [pl.BlockSpec((NC, blk, d), lambda i: (0, i, 0))]
        operands = (partials,)
    else:
        def body(p_ref, b_ref, o_ref):
            o_ref[...] = p_ref[0] + p_ref[1] + b_ref[...]
        in_specs = [pl.BlockSpec((NC, blk, d), lambda i: (0, i, 0)),
                    pl.BlockSpec((1, d), lambda i: (0, 0))]
        operands = (partials, bias.reshape(1, d))

    return pl.pallas_call(
        body,
        grid=(n_rows // blk,),
        in_specs=in_specs,
        out_specs=pl.BlockSpec((blk, d), lambda i: (i, 0)),
        out_shape=jax.ShapeDtypeStruct((n_rows, d), jnp.float32),
    )(*operands)


def _pad_edges(indices, values, e_pad):
    e = values.shape[0]
    rows, cols, vals = indices[0], indices[1], values
    if e != e_pad:
        pad = e_pad - e
        rows = jnp.concatenate([rows, jnp.zeros((pad,), jnp.int32)])
        cols = jnp.concatenate([cols, jnp.zeros((pad,), jnp.int32)])
        vals = jnp.concatenate([vals, jnp.zeros((pad,), jnp.float32)])
    return rows, cols, vals


def kernel(x, pt_indices, pt_values, pd_indices, pd_values, W, b):
    n, _ = x.shape
    d = W.shape[1]
    e = pt_values.shape[0]

    grain = NW * K * 2
    e_pad = -(-e // grain) * grain
    n_pad = -(-n // (NS * K)) * (NS * K)

    pt_rows, pt_cols, pt_vals = _pad_edges(pt_indices, pt_values, e_pad)
    pd_rows, pd_cols, pd_vals = _pad_edges(pd_indices, pd_values, e_pad)

    support = _matmul(x, W)
    p1 = _spmm_partials(pt_rows, pt_cols, pt_vals, support, n_pad, d)
    midpu = _combine(p1, None, n)
    p2 = _spmm_partials(pd_rows, pd_cols, pd_vals, midpu, n_pad, d)
    return _combine(p2, b, n)
